# baseline (speedup 1.0000x reference)
import functools

import jax
import jax.numpy as jnp
from jax import lax
from jax.experimental import pallas as pl
from jax.experimental.pallas import tpu as pltpu
from jax.experimental.pallas import tpu_sc as plsc

N_NODES = 10000
N_EDGES = 640000
EDGE_DIM = 16
HIDDEN = 128
HPACK = HIDDEN // 2
NUM_CLASSES = 10
NUM_GRAPHS = 64

NC, NS, L = 2, 16, 16
NPW = 640
NPAD = NS * NPW
HALF_E = N_EDGES // NC
SLAB = 1280
NSLAB = HALF_E // SLAB
GC = 64
NEG2 = -8323200

_BE = 5120


def _mlp_body(ea_ref, w1_ref, b1_ref, w2_ref, b2_ref, w3_ref, b3_ref, out_ref):
    h = jnp.maximum(
        jnp.dot(ea_ref[...].astype(jnp.bfloat16),
                w1_ref[...].astype(jnp.bfloat16),
                preferred_element_type=jnp.float32)
        + b1_ref[...], 0.0)
    h = jnp.maximum(
        jnp.dot(h.astype(jnp.bfloat16), w2_ref[...].astype(jnp.bfloat16),
                preferred_element_type=jnp.float32)
        + b2_ref[...], 0.0)
    msgs = (jnp.dot(h.astype(jnp.bfloat16), w3_ref[...].astype(jnp.bfloat16),
                    preferred_element_type=jnp.float32)
            + b3_ref[...])
    u = lax.bitcast_convert_type(msgs, jnp.int32)
    r = lax.shift_right_logical(
        u + 0x7FFF + (lax.shift_right_logical(u, 16) & 1), 16)
    lo = r[:, :HPACK] & 0xFFFF
    hi = lax.shift_left(r[:, HPACK:], 16)
    out_ref[...] = lo | hi


def _edge_mlp(ea, W1, b1, W2, b2, W3, b3):
    grid = (N_EDGES // _BE,)
    return pl.pallas_call(
        _mlp_body,
        grid=grid,
        in_specs=[
            pl.BlockSpec((_BE, EDGE_DIM), lambda i: (i, 0)),
            pl.BlockSpec((EDGE_DIM, HIDDEN), lambda i: (0, 0)),
            pl.BlockSpec((1, HIDDEN), lambda i: (0, 0)),
            pl.BlockSpec((HIDDEN, HIDDEN), lambda i: (0, 0)),
            pl.BlockSpec((1, HIDDEN), lambda i: (0, 0)),
            pl.BlockSpec((HIDDEN, HIDDEN), lambda i: (0, 0)),
            pl.BlockSpec((1, HIDDEN), lambda i: (0, 0)),
        ],
        out_specs=pl.BlockSpec((_BE, HPACK), lambda i: (i, 0)),
        out_shape=jax.ShapeDtypeStruct((N_EDGES, HPACK), jnp.int32),
    )(ea, W1, b1, W2, b2, W3, b3)


_sc_mesh = plsc.VectorSubcoreMesh(
    core_axis_name="c", subcore_axis_name="s", num_cores=NC, num_subcores=NS)


@functools.partial(
    pl.kernel,
    out_type=jax.ShapeDtypeStruct((NC, NPAD, HPACK), jnp.int32),
    mesh=_sc_mesh,
    scratch_types=[
        pltpu.VMEM((NPW + 4, HPACK), jnp.int32),
        pltpu.VMEM((SLAB,), jnp.int32),
        pltpu.VMEM((SLAB,), jnp.int32),
        pltpu.VMEM((SLAB + 32,), jnp.int32),
        pltpu.VMEM((SLAB + 32,), jnp.int32),
        pltpu.VMEM((SLAB + 32,), jnp.int32),
        pltpu.VMEM((SLAB + 32,), jnp.int32),
        pltpu.VMEM((GC, HPACK), jnp.int32),
        pltpu.VMEM((GC, HPACK), jnp.int32),
        pltpu.VMEM_SHARED((SLAB, HPACK), jnp.int32),
        pltpu.VMEM_SHARED((SLAB, HPACK), jnp.int32),
        pltpu.SemaphoreType.DMA,
        pltpu.SemaphoreType.DMA,
        pltpu.SemaphoreType.DMA,
        pltpu.SemaphoreType.DMA,
        pltpu.SemaphoreType.DMA,
        pltpu.SemaphoreType.DMA,
    ],
    compiler_params=pltpu.CompilerParams(needs_layout_passes=False),
)
def _scatter_max(dst_hbm, msgs_hbm, out_hbm,
                 acc, idxwa, idxwb, eidsa, dloca, eidsb, dlocb, rowsa, rowsb,
                 slaba, slabb, sema, semb, semia, semib, semga, semgb):
    cid = lax.axis_index("c")
    sid = lax.axis_index("s")
    base = sid * NPW
    ebase = cid * HALF_E
    lane = lax.iota(jnp.int32, L)

    neg = jnp.full((L,), NEG2, jnp.int32)

    def _init_row(r, carry):
        for j in range(HPACK // L):
            acc[r, pl.ds(j * L, L)] = neg
        return carry

    lax.fori_loop(0, NPW, _init_row, 0)

    zero = jnp.zeros((L,), jnp.int32)

    def _zero_eids(r, carry):
        eidsa[pl.ds(r * L, L)] = zero
        eidsb[pl.ds(r * L, L)] = zero
        return carry

    lax.fori_loop(0, (SLAB + 32) // L, _zero_eids, 0)

    scrap = jnp.full((L,), NPW, jnp.int32)

    def _issue_slab(t, sbuf, sem):
        pltpu.async_copy(msgs_hbm.at[pl.ds(ebase + t * SLAB, SLAB)], sbuf,
                         sem)

    def _wait_slab(t, sbuf, sem):
        pltpu.make_async_copy(msgs_hbm.at[pl.ds(ebase + t * SLAB, SLAB)],
                              sbuf, sem).wait()

    def _issue_idx(t, ibuf, sem):
        pltpu.async_copy(dst_hbm.at[pl.ds(ebase + t * SLAB, SLAB)], ibuf,
                         sem)

    def _wait_idx(t, ibuf, sem):
        pltpu.make_async_copy(dst_hbm.at[pl.ds(ebase + t * SLAB, SLAB)],
                              ibuf, sem).wait()

    def _accum_chunk(c, rowsr, nm, dlocr):
        rem = jnp.minimum(nm - c * GC, GC)
        ng = (rem + L - 1) // L

        def _grp(qq, carry2):
            gb = c * GC + qq * L
            dv = dlocr[pl.ds(gb, L)]
            for r in range(L):
                d = dv[r]
                rl = qq * L + r
                for j in range(HPACK // L):
                    sl = pl.ds(j * L, L)
                    a = plsc.bitcast(acc[d, sl], jnp.bfloat16)
                    g = plsc.bitcast(rowsr[rl, sl], jnp.bfloat16)
                    acc[d, sl] = plsc.bitcast(jnp.maximum(a, g), jnp.int32)
            return carry2

        lax.fori_loop(0, ng, _grp, 0)

    def _accum_prev(nm, eidsr, dlocr, sbuf, rowsr, semgr):
        nch = (nm + GC - 1) // GC

        @pl.when(nch > 0)
        def _drain():
            pltpu.make_async_copy(sbuf.at[eidsr.at[pl.ds(0, GC)]], rowsr,
                                  semgr).wait()
            _accum_chunk(0, rowsr, nm, dlocr)

            def _slow(c, carry):
                pltpu.async_copy(sbuf.at[eidsr.at[pl.ds(c * GC, GC)]],
                                 rowsr, semgr).wait()
                _accum_chunk(c, rowsr, nm, dlocr)
                return carry

            lax.fori_loop(1, nch, _slow, 0)

    _bufs = [
        dict(idx=idxwa, semi=semia, eids=eidsa, dloc=dloca, slab=slaba,
             sem=sema, rows=rowsa, semg=semga),
        dict(idx=idxwb, semi=semib, eids=eidsb, dloc=dlocb, slab=slabb,
             sem=semb, rows=rowsb, semg=semgb),
    ]

    def _stage(t, par, nm_prev):
        b = _bufs[par]
        o = _bufs[1 - par]
        _wait_idx(t, b["idx"], b["semi"])

        def _filter(s, ptr):
            v = b["idx"][pl.ds(s * L, L)]
            u = plsc.bitcast(v - base, jnp.uint32)
            m = u < jnp.uint32(NPW)
            pc = plsc.all_reduce_population_count(m)
            plsc.store_compressed(b["eids"].at[pl.ds(ptr, L)], s * L + lane,
                                  mask=m)
            plsc.store_compressed(b["dloc"].at[pl.ds(ptr, L)], v - base,
                                  mask=m)
            return ptr + pc[0]

        nm = lax.fori_loop(0, SLAB // L, _filter, 0, unroll=4)
        b["dloc"][pl.ds(nm, L)] = scrap

        @pl.when(t + 2 < NSLAB)
        def _issue_idx_next():
            _issue_idx(t + 2, b["idx"], b["semi"])

        _accum_prev(nm_prev, o["eids"], o["dloc"], o["slab"], o["rows"],
                    o["semg"])

        @pl.when(sid == 0)
        def _wait_msgs():
            _wait_slab(t, b["slab"], b["sem"])

        plsc.subcore_barrier()

        @pl.when((sid == 0) & (t + 1 < NSLAB))
        def _issue_msgs_next():
            _issue_slab(t + 1, o["slab"], o["sem"])

        @pl.when(nm > 0)
        def _issue_gather():
            pltpu.async_copy(b["slab"].at[b["eids"].at[pl.ds(0, GC)]],
                             b["rows"], b["semg"])

        return nm

    @pl.when(sid == 0)
    def _prime_slab():
        _issue_slab(0, slaba, sema)

    _issue_idx(0, idxwa, semia)
    _issue_idx(1, idxwb, semib)

    def _slab_pair(p, nm_prev):
        t0 = 2 * p
        nm1 = _stage(t0, 0, nm_prev)
        nm2 = _stage(t0 + 1, 1, nm1)
        return nm2

    nm_last = lax.fori_loop(0, NSLAB // 2, _slab_pair, 0)
    _accum_prev(nm_last, eidsb, dlocb, slabb, rowsb, semgb)

    pltpu.sync_copy(acc.at[pl.ds(0, NPW)], out_hbm.at[cid, pl.ds(base, NPW)])


def _pool_cls_body(xp_ref, m0_ref, batch_ref, wc1_ref, bc1_ref, wc2_ref,
                   bc2_ref, out_ref):
    def _unpack(p):
        flo = lax.bitcast_convert_type(lax.shift_left(p, 16), jnp.float32)
        fhi = lax.bitcast_convert_type(
            p & jnp.int32(-65536), jnp.float32)
        return jnp.concatenate([flo, fhi], axis=1)

    x = jnp.maximum(jnp.maximum(_unpack(xp_ref[0]), _unpack(xp_ref[1])),
                    m0_ref[...])
    gids = lax.broadcasted_iota(jnp.int32, (NUM_GRAPHS, NPAD), 0)
    mask = (gids == batch_ref[...]).astype(jnp.float32)
    sums = jnp.dot(mask, x, preferred_element_type=jnp.float32)
    counts = jnp.sum(mask, axis=1, keepdims=True)
    rep = sums / jnp.maximum(counts, 1.0)
    h = jnp.maximum(
        jnp.dot(rep, wc1_ref[...], preferred_element_type=jnp.float32)
        + bc1_ref[...], 0.0)
    out_ref[...] = (
        jnp.dot(h, wc2_ref[...], preferred_element_type=jnp.float32)
        + bc2_ref[...])


def _pool_cls(xp, msg0, batch2d, Wc1, bc1, Wc2, bc2):
    return pl.pallas_call(
        _pool_cls_body,
        out_shape=jax.ShapeDtypeStruct((NUM_GRAPHS, NUM_CLASSES), jnp.float32),
    )(xp, msg0, batch2d, Wc1, bc1, Wc2, bc2)


def kernel(edge_index, edge_attr, batch, W1, b1, W2, b2, W3, b3, Wc1, bc1,
           Wc2, bc2):
    dst = edge_index[1].astype(jnp.int32)
    msgs = _edge_mlp(edge_attr, W1, b1.reshape(1, HIDDEN), W2,
                     b2.reshape(1, HIDDEN), W3, b3.reshape(1, HIDDEN))
    h0 = jnp.maximum(jnp.maximum(b1, 0.0) @ W2 + b2, 0.0)
    msg0 = (h0 @ W3 + b3).reshape(1, HIDDEN)
    xp = _scatter_max(dst, msgs)
    batch_p = jnp.pad(batch.astype(jnp.int32), (0, NPAD - N_NODES),
                      constant_values=NUM_GRAPHS).reshape(1, NPAD)
    return _pool_cls(xp, msg0, batch_p, Wc1, bc1.reshape(1, HIDDEN), Wc2,
                     bc2.reshape(1, NUM_CLASSES))


# speedup vs baseline: 1.0242x; 1.0242x over previous
import functools

import jax
import jax.numpy as jnp
from jax import lax
from jax.experimental import pallas as pl
from jax.experimental.pallas import tpu as pltpu
from jax.experimental.pallas import tpu_sc as plsc

N_NODES = 10000
N_EDGES = 640000
EDGE_DIM = 16
HIDDEN = 128
HPACK = HIDDEN // 2
NUM_CLASSES = 10
NUM_GRAPHS = 64

NC, NS, L = 2, 16, 16
NPW = 640
NPAD = NS * NPW
HALF_E = N_EDGES // NC
SLAB = 1280
NSLAB = HALF_E // SLAB
GC = 64
NEG2 = -8323200

_BE = 5120


def _mlp_body(ea_ref, w1_ref, b1_ref, w2_ref, b2_ref, w3_ref, b3_ref, out_ref):
    h = jnp.maximum(
        jnp.dot(ea_ref[...].astype(jnp.bfloat16),
                w1_ref[...].astype(jnp.bfloat16),
                preferred_element_type=jnp.float32)
        + b1_ref[...], 0.0)
    h = jnp.maximum(
        jnp.dot(h.astype(jnp.bfloat16), w2_ref[...].astype(jnp.bfloat16),
                preferred_element_type=jnp.float32)
        + b2_ref[...], 0.0)
    msgs = (jnp.dot(h.astype(jnp.bfloat16), w3_ref[...].astype(jnp.bfloat16),
                    preferred_element_type=jnp.float32)
            + b3_ref[...])
    u = lax.bitcast_convert_type(msgs, jnp.int32)
    r = lax.shift_right_logical(
        u + 0x7FFF + (lax.shift_right_logical(u, 16) & 1), 16)
    lo = r[:, :HPACK] & 0xFFFF
    hi = lax.shift_left(r[:, HPACK:], 16)
    out_ref[...] = lo | hi


def _edge_mlp(ea, W1, b1, W2, b2, W3, b3):
    grid = (N_EDGES // _BE,)
    return pl.pallas_call(
        _mlp_body,
        grid=grid,
        in_specs=[
            pl.BlockSpec((_BE, EDGE_DIM), lambda i: (i, 0)),
            pl.BlockSpec((EDGE_DIM, HIDDEN), lambda i: (0, 0)),
            pl.BlockSpec((1, HIDDEN), lambda i: (0, 0)),
            pl.BlockSpec((HIDDEN, HIDDEN), lambda i: (0, 0)),
            pl.BlockSpec((1, HIDDEN), lambda i: (0, 0)),
            pl.BlockSpec((HIDDEN, HIDDEN), lambda i: (0, 0)),
            pl.BlockSpec((1, HIDDEN), lambda i: (0, 0)),
        ],
        out_specs=pl.BlockSpec((_BE, HPACK), lambda i: (i, 0)),
        out_shape=jax.ShapeDtypeStruct((N_EDGES, HPACK), jnp.int32),
    )(ea, W1, b1, W2, b2, W3, b3)


_sc_mesh = plsc.VectorSubcoreMesh(
    core_axis_name="c", subcore_axis_name="s", num_cores=NC, num_subcores=NS)


@functools.partial(
    pl.kernel,
    out_type=jax.ShapeDtypeStruct((NC, NPAD, HPACK), jnp.int32),
    mesh=_sc_mesh,
    scratch_types=[
        pltpu.VMEM((NPW + 4, HPACK), jnp.int32),
        pltpu.VMEM((SLAB,), jnp.int32),
        pltpu.VMEM((SLAB,), jnp.int32),
        pltpu.VMEM((SLAB + 32,), jnp.int32),
        pltpu.VMEM((SLAB + 32,), jnp.int32),
        pltpu.VMEM((GC, HPACK), jnp.int32),
        pltpu.VMEM((GC, HPACK), jnp.int32),
        pltpu.VMEM_SHARED((SLAB, HPACK), jnp.int32),
        pltpu.VMEM_SHARED((SLAB, HPACK), jnp.int32),
        pltpu.SemaphoreType.DMA,
        pltpu.SemaphoreType.DMA,
        pltpu.SemaphoreType.DMA,
        pltpu.SemaphoreType.DMA,
        pltpu.SemaphoreType.DMA,
        pltpu.SemaphoreType.DMA,
    ],
    compiler_params=pltpu.CompilerParams(needs_layout_passes=False),
)
def _scatter_max(dst_hbm, msgs_hbm, out_hbm,
                 acc, idxwa, idxwb, eids, dloc, rowsa, rowsb,
                 slaba, slabb, sema, semb, semia, semib, semga, semgb):
    cid = lax.axis_index("c")
    sid = lax.axis_index("s")
    base = sid * NPW
    ebase = cid * HALF_E
    lane = lax.iota(jnp.int32, L)

    neg = jnp.full((L,), NEG2, jnp.int32)

    def _init_row(r, carry):
        for j in range(HPACK // L):
            acc[r, pl.ds(j * L, L)] = neg
        return carry

    lax.fori_loop(0, NPW, _init_row, 0)

    zero = jnp.zeros((L,), jnp.int32)

    def _zero_eids(r, carry):
        eids[pl.ds(r * L, L)] = zero
        return carry

    lax.fori_loop(0, (SLAB + 32) // L, _zero_eids, 0)

    scrap = jnp.full((L,), NPW, jnp.int32)

    def _issue_slab(t, sbuf, sem):
        pltpu.async_copy(msgs_hbm.at[pl.ds(ebase + t * SLAB, SLAB)], sbuf,
                         sem)

    def _wait_slab(t, sbuf, sem):
        pltpu.make_async_copy(msgs_hbm.at[pl.ds(ebase + t * SLAB, SLAB)],
                              sbuf, sem).wait()

    def _issue_idx(t, ibuf, sem):
        pltpu.async_copy(dst_hbm.at[pl.ds(ebase + t * SLAB, SLAB)], ibuf,
                         sem)

    def _wait_idx(t, ibuf, sem):
        pltpu.make_async_copy(dst_hbm.at[pl.ds(ebase + t * SLAB, SLAB)],
                              ibuf, sem).wait()

    def _accum(c, rowsr, nm):
        rem = jnp.minimum(nm - c * GC, GC)
        ng = (rem + L - 1) // L

        def _grp(qq, carry2):
            gb = c * GC + qq * L
            dv = dloc[pl.ds(gb, L)]
            for r in range(L):
                d = dv[r]
                rl = qq * L + r
                for j in range(HPACK // L):
                    sl = pl.ds(j * L, L)
                    a = plsc.bitcast(acc[d, sl], jnp.bfloat16)
                    g = plsc.bitcast(rowsr[rl, sl], jnp.bfloat16)
                    acc[d, sl] = plsc.bitcast(jnp.maximum(a, g), jnp.int32)
            return carry2

        lax.fori_loop(0, ng, _grp, 0)

    def _do_slab(t, sbuf, idxr):
        def _filter(s, ptr):
            v = idxr[pl.ds(s * L, L)]
            u = plsc.bitcast(v - base, jnp.uint32)
            m = u < jnp.uint32(NPW)
            pc = plsc.all_reduce_population_count(m)
            plsc.store_compressed(eids.at[pl.ds(ptr, L)], s * L + lane,
                                  mask=m)
            plsc.store_compressed(dloc.at[pl.ds(ptr, L)], v - base, mask=m)
            return ptr + pc[0]

        nm = lax.fori_loop(0, SLAB // L, _filter, 0, unroll=4)
        dloc[pl.ds(nm, L)] = scrap
        nch = (nm + GC - 1) // GC

        def _gather_src(c):
            return sbuf.at[eids.at[pl.ds(c * GC, GC)]]

        @pl.when(nch > 0)
        def _prime():
            pltpu.async_copy(_gather_src(0), rowsa, semga)

        def _chunk_pair(q, carry):
            c0 = 2 * q
            pltpu.make_async_copy(_gather_src(c0), rowsa, semga).wait()

            @pl.when(c0 + 1 < nch)
            def _issue_b():
                pltpu.async_copy(_gather_src(c0 + 1), rowsb, semgb)

            _accum(c0, rowsa, nm)

            @pl.when(c0 + 1 < nch)
            def _do_b():
                pltpu.make_async_copy(_gather_src(c0 + 1), rowsb,
                                      semgb).wait()

                @pl.when(c0 + 2 < nch)
                def _issue_a():
                    pltpu.async_copy(_gather_src(c0 + 2), rowsa, semga)

                _accum(c0 + 1, rowsb, nm)

            return carry

        lax.fori_loop(0, (nch + 1) // 2, _chunk_pair, 0)

    @pl.when(sid == 0)
    def _prime_slab():
        _issue_slab(0, slaba, sema)

    _issue_idx(0, idxwa, semia)

    def _slab_pair(p, carry):
        t0 = 2 * p

        @pl.when(sid == 0)
        def _wait_a():
            _wait_slab(t0, slaba, sema)

        plsc.subcore_barrier()

        @pl.when(sid == 0)
        def _issue_next_b():
            _issue_slab(t0 + 1, slabb, semb)

        _wait_idx(t0, idxwa, semia)
        _issue_idx(t0 + 1, idxwb, semib)
        _do_slab(t0, slaba, idxwa)

        @pl.when(sid == 0)
        def _wait_b():
            _wait_slab(t0 + 1, slabb, semb)

        plsc.subcore_barrier()

        @pl.when((sid == 0) & (t0 + 2 < NSLAB))
        def _issue_next_a():
            _issue_slab(t0 + 2, slaba, sema)

        _wait_idx(t0 + 1, idxwb, semib)

        @pl.when(t0 + 2 < NSLAB)
        def _issue_idx_a():
            _issue_idx(t0 + 2, idxwa, semia)

        _do_slab(t0 + 1, slabb, idxwb)
        return carry

    lax.fori_loop(0, NSLAB // 2, _slab_pair, 0)

    pltpu.sync_copy(acc.at[pl.ds(0, NPW)], out_hbm.at[cid, pl.ds(base, NPW)])


def _pool_cls_body(xp_ref, m0_ref, batch_ref, wc1_ref, bc1_ref, wc2_ref,
                   bc2_ref, out_ref):
    def _unpack(p):
        flo = lax.bitcast_convert_type(lax.shift_left(p, 16), jnp.float32)
        fhi = lax.bitcast_convert_type(
            p & jnp.int32(-65536), jnp.float32)
        return jnp.concatenate([flo, fhi], axis=1)

    x = jnp.maximum(jnp.maximum(_unpack(xp_ref[0]), _unpack(xp_ref[1])),
                    m0_ref[...])
    gids = lax.broadcasted_iota(jnp.int32, (NUM_GRAPHS, NPAD), 0)
    mask = (gids == batch_ref[...]).astype(jnp.float32)
    sums = jnp.dot(mask, x, preferred_element_type=jnp.float32)
    counts = jnp.sum(mask, axis=1, keepdims=True)
    rep = sums / jnp.maximum(counts, 1.0)
    h = jnp.maximum(
        jnp.dot(rep, wc1_ref[...], preferred_element_type=jnp.float32)
        + bc1_ref[...], 0.0)
    out_ref[...] = (
        jnp.dot(h, wc2_ref[...], preferred_element_type=jnp.float32)
        + bc2_ref[...])


def _pool_cls(xp, msg0, batch2d, Wc1, bc1, Wc2, bc2):
    return pl.pallas_call(
        _pool_cls_body,
        out_shape=jax.ShapeDtypeStruct((NUM_GRAPHS, NUM_CLASSES), jnp.float32),
    )(xp, msg0, batch2d, Wc1, bc1, Wc2, bc2)


def kernel(edge_index, edge_attr, batch, W1, b1, W2, b2, W3, b3, Wc1, bc1,
           Wc2, bc2):
    dst = edge_index[1].astype(jnp.int32)
    msgs = _edge_mlp(edge_attr, W1, b1.reshape(1, HIDDEN), W2,
                     b2.reshape(1, HIDDEN), W3, b3.reshape(1, HIDDEN))
    h0 = jnp.maximum(jnp.maximum(b1, 0.0) @ W2 + b2, 0.0)
    msg0 = (h0 @ W3 + b3).reshape(1, HIDDEN)
    xp = _scatter_max(dst, msgs)
    batch_p = jnp.pad(batch.astype(jnp.int32), (0, NPAD - N_NODES),
                      constant_values=NUM_GRAPHS).reshape(1, NPAD)
    return _pool_cls(xp, msg0, batch_p, Wc1, bc1.reshape(1, HIDDEN), Wc2,
                     bc2.reshape(1, NUM_CLASSES))
